# baseline (speedup 1.0000x reference)
import functools

import jax
import jax.numpy as jnp
from jax import lax
from jax.experimental import pallas as pl
from jax.experimental.pallas import tpu as pltpu
from jax.experimental.pallas import tpu_sc as plsc

_B, _N, _NBH = 8, 512, 32
_AB, _SB, _NF = 256, 64, 256
_BN = 256
_NBLK = _N // _BN
_E = _BN * _NBH
_NE = _B * _N * _NBH

_BF = jnp.bfloat16

_LOG2E = 1.4426950408889634
_LN2 = 0.6931471805599453


def _ssp_scaled(z):
    t = jnp.exp2(-jnp.abs(z))
    return (jnp.log2(1.0 + t) - 1.0 - jnp.minimum(z, 0.0)) * _LN2


def _y_kernel(x_ref, Wi_ref, y_ref):
    b = pl.program_id(0)

    @pl.when(b < _B)
    def _():
        y_ref[0] = jnp.dot(x_ref[0].astype(_BF), Wi_ref[:],
                           preferred_element_type=jnp.float32)

    @pl.when(b == _B)
    def _():
        y_ref[0] = jnp.zeros((_N, _NF), jnp.float32)


_SC_INFO = plsc.get_sparse_core_info()
_NW = _SC_INFO.num_cores * _SC_INFO.num_subcores
_PER_W = _NE // _NW
_CH = 64
_NCH = _PER_W // _CH


def _sc_gather(table_hbm, idx_hbm, out_hbm, idx_v, rows_v, sem):
    wid = lax.axis_index("s") * _SC_INFO.num_cores + lax.axis_index("c")
    base = wid * _PER_W

    def body(i, carry):
        off = base + i * _CH
        pltpu.sync_copy(idx_hbm.at[pl.ds(off, _CH)], idx_v)
        pltpu.async_copy(table_hbm.at[idx_v], rows_v, sem).wait()
        pltpu.sync_copy(rows_v, out_hbm.at[pl.ds(off, _CH)])
        return carry

    lax.fori_loop(0, _NCH, body, 0)


def _block_kernel(ynbh_ref, f_ref,
                  W1_ref, b1_ref, W2_ref, b2_ref,
                  Wf_ref, bf_ref, Wd_ref, bd_ref,
                  out_ref):
    f = f_ref[0].reshape(_E, _SB)
    h = _ssp_scaled(jnp.dot(f.astype(_BF), W1_ref[:],
                            preferred_element_type=jnp.float32) + b1_ref[:])
    wfilt = jnp.dot(h.astype(_BF), W2_ref[:],
                    preferred_element_type=jnp.float32) + b2_ref[:]

    agg = (ynbh_ref[0, 0] * wfilt).reshape(_BN, _NBH, _NF).sum(axis=1)
    v = _ssp_scaled(jnp.dot(agg.astype(_BF), Wf_ref[:],
                            preferred_element_type=jnp.float32) + bf_ref[:])
    out_ref[0] = jnp.dot(v.astype(_BF), Wd_ref[:],
                         preferred_element_type=jnp.float32) + bd_ref[:]


def kernel(x, r_ij, neighbors, neighbor_mask, f_ij,
           W1, b1, W2, b2, Wi, Wf, bf, Wd, bd):
    del r_ij

    y = pl.pallas_call(
        _y_kernel,
        grid=(_B + 1,),
        in_specs=[
            pl.BlockSpec((1, _N, _AB), lambda b: (jnp.minimum(b, _B - 1), 0, 0)),
            pl.BlockSpec((_AB, _NF), lambda b: (0, 0)),
        ],
        out_specs=pl.BlockSpec((1, _N, _NF), lambda b: (b, 0, 0)),
        out_shape=jax.ShapeDtypeStruct((_B + 1, _N, _NF), jnp.float32),
    )(x, Wi.astype(_BF))
    table = y.reshape((_B + 1) * _N, _NF)

    boff = (jnp.arange(_B, dtype=jnp.int32) * _N)[:, None, None]
    gidx = jnp.where(neighbor_mask > 0, neighbors + boff, _B * _N)
    gidx = gidx.reshape(_NE)

    sc_gather = functools.partial(
        pl.kernel,
        mesh=plsc.VectorSubcoreMesh(core_axis_name="c", subcore_axis_name="s"),
        out_type=jax.ShapeDtypeStruct((_NE, _NF), jnp.float32),
        scratch_types=[
            pltpu.VMEM((_CH,), jnp.int32),
            pltpu.VMEM((_CH, _NF), jnp.float32),
            pltpu.SemaphoreType.DMA,
        ],
    )(_sc_gather)
    y_nbh = sc_gather(table, gidx)
    ynbh_r = y_nbh.reshape(_B, _NBLK, _E, _NF)

    full = lambda shape: pl.BlockSpec(shape, lambda b, nb: (0,) * len(shape))

    out = pl.pallas_call(
        _block_kernel,
        grid=(_B, _NBLK),
        in_specs=[
            pl.BlockSpec((1, 1, _E, _NF), lambda b, nb: (b, nb, 0, 0)),
            pl.BlockSpec((1, _BN, _NBH, _SB), lambda b, nb: (b, nb, 0, 0)),
            full((_SB, _NF)),
            full((1, _NF)),
            full((_NF, _NF)),
            full((1, _NF)),
            full((_NF, _AB)),
            full((1, _AB)),
            full((_AB, _AB)),
            full((1, _AB)),
        ],
        out_specs=pl.BlockSpec((1, _BN, _AB), lambda b, nb: (b, nb, 0)),
        out_shape=jax.ShapeDtypeStruct((_B, _N, _AB), jnp.float32),
        compiler_params=pltpu.CompilerParams(
            dimension_semantics=("parallel", "parallel"),
        ),
    )(ynbh_r, f_ij,
      (W1 * -_LOG2E).astype(_BF), (b1 * -_LOG2E).reshape(1, _NF),
      W2.astype(_BF), b2.reshape(1, _NF),
      (Wf * -_LOG2E).astype(_BF), (bf * -_LOG2E).reshape(1, _AB),
      Wd.astype(_BF), bd.reshape(1, _AB))
    return out


# speedup vs baseline: 1.2563x; 1.2563x over previous
import functools

import jax
import jax.numpy as jnp
from jax import lax
from jax.experimental import pallas as pl
from jax.experimental.pallas import tpu as pltpu
from jax.experimental.pallas import tpu_sc as plsc

_B, _N, _NBH = 8, 512, 32
_AB, _SB, _NF = 256, 64, 256
_BN = 256
_NBLK = _N // _BN
_E = _BN * _NBH
_NE = _B * _N * _NBH

_BF = jnp.bfloat16

_LOG2E = 1.4426950408889634
_LN2 = 0.6931471805599453


def _ssp_scaled(z):
    t = jnp.exp2(-jnp.abs(z))
    return (jnp.log2(1.0 + t) - 1.0 - jnp.minimum(z, 0.0)) * _LN2


def _y_kernel(x_ref, Wi_ref, y_ref):
    b = pl.program_id(0)

    @pl.when(b < _B)
    def _():
        yv = jnp.dot(x_ref[0].astype(_BF), Wi_ref[:],
                     preferred_element_type=jnp.float32)
        wl = lax.bitcast_convert_type(yv[:, :_NF // 2], jnp.uint32)
        wh = lax.bitcast_convert_type(yv[:, _NF // 2:], jnp.uint32)
        wl = (wl + jnp.uint32(0x8000)) >> jnp.uint32(16)
        wh = (wh + jnp.uint32(0x8000)) & jnp.uint32(0xFFFF0000)
        y_ref[0] = lax.bitcast_convert_type(wh | wl, jnp.int32)

    @pl.when(b == _B)
    def _():
        y_ref[0] = jnp.zeros((_N, _NF // 2), jnp.int32)


_SC_INFO = plsc.get_sparse_core_info()
_NW = _SC_INFO.num_cores * _SC_INFO.num_subcores
_PER_W = _NE // _NW
_CH = 64
_NCH = _PER_W // _CH


def _sc_gather(table_hbm, idx_hbm, out_hbm, idx_v, rows_v, sem):
    wid = lax.axis_index("s") * _SC_INFO.num_cores + lax.axis_index("c")
    base = wid * _PER_W

    def body(i, carry):
        off = base + i * _CH
        pltpu.sync_copy(idx_hbm.at[pl.ds(off, _CH)], idx_v)
        pltpu.async_copy(table_hbm.at[idx_v], rows_v, sem).wait()
        pltpu.sync_copy(rows_v, out_hbm.at[pl.ds(off, _CH)])
        return carry

    lax.fori_loop(0, _NCH, body, 0)


def _filt_kernel(f_ref, W1_ref, b1_ref, W2_ref, b2_ref, wf_ref):
    f = f_ref[0].reshape(_E, _SB)
    h = _ssp_scaled(jnp.dot(f.astype(_BF), W1_ref[:],
                            preferred_element_type=jnp.float32) + b1_ref[:])
    wf_ref[0, 0] = (jnp.dot(h.astype(_BF), W2_ref[:],
                            preferred_element_type=jnp.float32)
                    + b2_ref[:]).astype(_BF)


def _comb_kernel(ynbh_ref, wf_ref, Wf_ref, bf_ref, Wd_ref, bd_ref, out_ref):
    w = lax.bitcast_convert_type(ynbh_ref[0, 0], jnp.uint32)
    y_lo = lax.bitcast_convert_type(w << jnp.uint32(16), jnp.float32)
    y_hi = lax.bitcast_convert_type(w & jnp.uint32(0xFFFF0000), jnp.float32)
    wf = wf_ref[0, 0].astype(jnp.float32)
    agg_lo = (y_lo * wf[:, :_NF // 2]).reshape(_BN, _NBH, _NF // 2).sum(axis=1)
    agg_hi = (y_hi * wf[:, _NF // 2:]).reshape(_BN, _NBH, _NF // 2).sum(axis=1)
    agg = jnp.concatenate([agg_lo, agg_hi], axis=1)
    v = _ssp_scaled(jnp.dot(agg.astype(_BF), Wf_ref[:],
                            preferred_element_type=jnp.float32) + bf_ref[:])
    out_ref[0] = jnp.dot(v.astype(_BF), Wd_ref[:],
                         preferred_element_type=jnp.float32) + bd_ref[:]


def kernel(x, r_ij, neighbors, neighbor_mask, f_ij,
           W1, b1, W2, b2, Wi, Wf, bf, Wd, bd):
    del r_ij

    y = pl.pallas_call(
        _y_kernel,
        grid=(_B + 1,),
        in_specs=[
            pl.BlockSpec((1, _N, _AB), lambda b: (jnp.minimum(b, _B - 1), 0, 0)),
            pl.BlockSpec((_AB, _NF), lambda b: (0, 0)),
        ],
        out_specs=pl.BlockSpec((1, _N, _NF // 2), lambda b: (b, 0, 0)),
        out_shape=jax.ShapeDtypeStruct((_B + 1, _N, _NF // 2), jnp.int32),
    )(x, Wi.astype(_BF))
    table = y.reshape((_B + 1) * _N, _NF // 2)

    boff = (jnp.arange(_B, dtype=jnp.int32) * _N)[:, None, None]
    gidx = jnp.where(neighbor_mask > 0, neighbors + boff, _B * _N)
    gidx = gidx.reshape(_NE)

    sc_gather = functools.partial(
        pl.kernel,
        mesh=plsc.VectorSubcoreMesh(core_axis_name="c", subcore_axis_name="s"),
        out_type=jax.ShapeDtypeStruct((_NE, _NF // 2), jnp.int32),
        scratch_types=[
            pltpu.VMEM((_CH,), jnp.int32),
            pltpu.VMEM((_CH, _NF // 2), jnp.int32),
            pltpu.SemaphoreType.DMA,
        ],
    )(_sc_gather)
    y_nbh = sc_gather(table, gidx)
    ynbh_r = y_nbh.reshape(_B, _NBLK, _E, _NF // 2)

    full = lambda shape: pl.BlockSpec(shape, lambda b, nb: (0,) * len(shape))

    wfilt = pl.pallas_call(
        _filt_kernel,
        grid=(_B, _NBLK),
        in_specs=[
            pl.BlockSpec((1, _BN, _NBH, _SB), lambda b, nb: (b, nb, 0, 0)),
            full((_SB, _NF)),
            full((1, _NF)),
            full((_NF, _NF)),
            full((1, _NF)),
        ],
        out_specs=pl.BlockSpec((1, 1, _E, _NF), lambda b, nb: (b, nb, 0, 0)),
        out_shape=jax.ShapeDtypeStruct((_B, _NBLK, _E, _NF), _BF),
        compiler_params=pltpu.CompilerParams(
            dimension_semantics=("parallel", "parallel"),
        ),
    )(f_ij,
      (W1 * -_LOG2E).astype(_BF), (b1 * -_LOG2E).reshape(1, _NF),
      W2.astype(_BF), b2.reshape(1, _NF))

    out = pl.pallas_call(
        _comb_kernel,
        grid=(_B, _NBLK),
        in_specs=[
            pl.BlockSpec((1, 1, _E, _NF // 2), lambda b, nb: (b, nb, 0, 0)),
            pl.BlockSpec((1, 1, _E, _NF), lambda b, nb: (b, nb, 0, 0)),
            full((_NF, _AB)),
            full((1, _AB)),
            full((_AB, _AB)),
            full((1, _AB)),
        ],
        out_specs=pl.BlockSpec((1, _BN, _AB), lambda b, nb: (b, nb, 0)),
        out_shape=jax.ShapeDtypeStruct((_B, _N, _AB), jnp.float32),
        compiler_params=pltpu.CompilerParams(
            dimension_semantics=("parallel", "parallel"),
        ),
    )(ynbh_r, wfilt,
      (Wf * -_LOG2E).astype(_BF), (bf * -_LOG2E).reshape(1, _AB),
      Wd.astype(_BF), bd.reshape(1, _AB))
    return out


# speedup vs baseline: 1.2809x; 1.0195x over previous
import functools

import jax
import jax.numpy as jnp
from jax import lax
from jax.experimental import pallas as pl
from jax.experimental.pallas import tpu as pltpu
from jax.experimental.pallas import tpu_sc as plsc

_B, _N, _NBH = 8, 512, 32
_AB, _SB, _NF = 256, 64, 256
_BN = 256
_NBLK = _N // _BN
_E = _BN * _NBH
_NE = _B * _N * _NBH

_BF = jnp.bfloat16

_LOG2E = 1.4426950408889634
_LN2 = 0.6931471805599453


def _ssp_scaled(z):
    t = jnp.exp2(-jnp.abs(z))
    return (jnp.log2(1.0 + t) - 1.0 - jnp.minimum(z, 0.0)) * _LN2


def _y_kernel(x_ref, Wi_ref, y_ref):
    b = pl.program_id(0)

    @pl.when(b < _B)
    def _():
        yv = jnp.dot(x_ref[0].astype(_BF), Wi_ref[:],
                     preferred_element_type=jnp.float32)
        wl = lax.bitcast_convert_type(yv[:, :_NF // 2], jnp.uint32)
        wh = lax.bitcast_convert_type(yv[:, _NF // 2:], jnp.uint32)
        wl = (wl + jnp.uint32(0x8000)) >> jnp.uint32(16)
        wh = (wh + jnp.uint32(0x8000)) & jnp.uint32(0xFFFF0000)
        y_ref[0] = lax.bitcast_convert_type(wh | wl, jnp.int32)

    @pl.when(b == _B)
    def _():
        y_ref[0] = jnp.zeros((_N, _NF // 2), jnp.int32)


_SC_INFO = plsc.get_sparse_core_info()
_NW = _SC_INFO.num_cores * _SC_INFO.num_subcores
_PER_W = _NE // _NW
_CH = 64
_NCH = _PER_W // _CH


def _sc_gather(table_hbm, idx_hbm, out_hbm, idx_v, rows0, rows1, sem0, sem1):
    wid = lax.axis_index("s") * _SC_INFO.num_cores + lax.axis_index("c")
    base = wid * _PER_W
    pltpu.sync_copy(idx_hbm.at[pl.ds(base, _PER_W)], idx_v)

    bufs = (rows0, rows1)
    sems = (sem0, sem1)
    cp_prev = None
    for c in range(_NCH):
        cp = pltpu.async_copy(table_hbm.at[idx_v.at[pl.ds(c * _CH, _CH)]],
                              bufs[c % 2], sems[c % 2])
        if cp_prev is not None:
            cp_prev.wait()
            pltpu.sync_copy(bufs[(c - 1) % 2],
                            out_hbm.at[pl.ds(base + (c - 1) * _CH, _CH)])
        cp_prev = cp
    cp_prev.wait()
    pltpu.sync_copy(bufs[(_NCH - 1) % 2],
                    out_hbm.at[pl.ds(base + (_NCH - 1) * _CH, _CH)])


def _filt_kernel(f_ref, W1_ref, b1_ref, W2_ref, b2_ref, wf_ref):
    f = f_ref[0].reshape(_E, _SB)
    h = _ssp_scaled(jnp.dot(f.astype(_BF), W1_ref[:],
                            preferred_element_type=jnp.float32) + b1_ref[:])
    wf_ref[0, 0] = (jnp.dot(h.astype(_BF), W2_ref[:],
                            preferred_element_type=jnp.float32)
                    + b2_ref[:]).astype(_BF)


def _comb_kernel(ynbh_ref, wf_ref, Wf_ref, bf_ref, Wd_ref, bd_ref, out_ref):
    w = lax.bitcast_convert_type(ynbh_ref[0, 0], jnp.uint32)
    y_lo = lax.bitcast_convert_type(w << jnp.uint32(16), jnp.float32)
    y_hi = lax.bitcast_convert_type(w & jnp.uint32(0xFFFF0000), jnp.float32)
    wf = wf_ref[0, 0].astype(jnp.float32)
    agg_lo = (y_lo * wf[:, :_NF // 2]).reshape(_BN, _NBH, _NF // 2).sum(axis=1)
    agg_hi = (y_hi * wf[:, _NF // 2:]).reshape(_BN, _NBH, _NF // 2).sum(axis=1)
    agg = jnp.concatenate([agg_lo, agg_hi], axis=1)
    v = _ssp_scaled(jnp.dot(agg.astype(_BF), Wf_ref[:],
                            preferred_element_type=jnp.float32) + bf_ref[:])
    out_ref[0] = jnp.dot(v.astype(_BF), Wd_ref[:],
                         preferred_element_type=jnp.float32) + bd_ref[:]


def kernel(x, r_ij, neighbors, neighbor_mask, f_ij,
           W1, b1, W2, b2, Wi, Wf, bf, Wd, bd):
    del r_ij

    y = pl.pallas_call(
        _y_kernel,
        grid=(_B + 1,),
        in_specs=[
            pl.BlockSpec((1, _N, _AB), lambda b: (jnp.minimum(b, _B - 1), 0, 0)),
            pl.BlockSpec((_AB, _NF), lambda b: (0, 0)),
        ],
        out_specs=pl.BlockSpec((1, _N, _NF // 2), lambda b: (b, 0, 0)),
        out_shape=jax.ShapeDtypeStruct((_B + 1, _N, _NF // 2), jnp.int32),
    )(x, Wi.astype(_BF))
    table = y.reshape((_B + 1) * _N, _NF // 2)

    boff = (jnp.arange(_B, dtype=jnp.int32) * _N)[:, None, None]
    gidx = jnp.where(neighbor_mask > 0, neighbors + boff, _B * _N)
    gidx = gidx.reshape(_NE)

    sc_gather = functools.partial(
        pl.kernel,
        mesh=plsc.VectorSubcoreMesh(core_axis_name="c", subcore_axis_name="s"),
        out_type=jax.ShapeDtypeStruct((_NE, _NF // 2), jnp.int32),
        scratch_types=[
            pltpu.VMEM((_PER_W,), jnp.int32),
            pltpu.VMEM((_CH, _NF // 2), jnp.int32),
            pltpu.VMEM((_CH, _NF // 2), jnp.int32),
            pltpu.SemaphoreType.DMA,
            pltpu.SemaphoreType.DMA,
        ],
    )(_sc_gather)
    y_nbh = sc_gather(table, gidx)
    ynbh_r = y_nbh.reshape(_B, _NBLK, _E, _NF // 2)

    full = lambda shape: pl.BlockSpec(shape, lambda b, nb: (0,) * len(shape))

    wfilt = pl.pallas_call(
        _filt_kernel,
        grid=(_B, _NBLK),
        in_specs=[
            pl.BlockSpec((1, _BN, _NBH, _SB), lambda b, nb: (b, nb, 0, 0)),
            full((_SB, _NF)),
            full((1, _NF)),
            full((_NF, _NF)),
            full((1, _NF)),
        ],
        out_specs=pl.BlockSpec((1, 1, _E, _NF), lambda b, nb: (b, nb, 0, 0)),
        out_shape=jax.ShapeDtypeStruct((_B, _NBLK, _E, _NF), _BF),
        compiler_params=pltpu.CompilerParams(
            dimension_semantics=("parallel", "parallel"),
        ),
    )(f_ij,
      (W1 * -_LOG2E).astype(_BF), (b1 * -_LOG2E).reshape(1, _NF),
      W2.astype(_BF), b2.reshape(1, _NF))

    out = pl.pallas_call(
        _comb_kernel,
        grid=(_B, _NBLK),
        in_specs=[
            pl.BlockSpec((1, 1, _E, _NF // 2), lambda b, nb: (b, nb, 0, 0)),
            pl.BlockSpec((1, 1, _E, _NF), lambda b, nb: (b, nb, 0, 0)),
            full((_NF, _AB)),
            full((1, _AB)),
            full((_AB, _AB)),
            full((1, _AB)),
        ],
        out_specs=pl.BlockSpec((1, _BN, _AB), lambda b, nb: (b, nb, 0)),
        out_shape=jax.ShapeDtypeStruct((_B, _N, _AB), jnp.float32),
        compiler_params=pltpu.CompilerParams(
            dimension_semantics=("parallel", "parallel"),
        ),
    )(ynbh_r, wfilt,
      (Wf * -_LOG2E).astype(_BF), (bf * -_LOG2E).reshape(1, _AB),
      Wd.astype(_BF), bd.reshape(1, _AB))
    return out


# speedup vs baseline: 1.4879x; 1.1616x over previous
import functools

import jax
import jax.numpy as jnp
from jax import lax
from jax.experimental import pallas as pl
from jax.experimental.pallas import tpu as pltpu
from jax.experimental.pallas import tpu_sc as plsc

_B, _N, _NBH = 8, 512, 32
_AB, _SB, _NF = 256, 64, 256
_BN = 256
_NBLK = _N // _BN
_E = _BN * _NBH
_NE = _B * _N * _NBH

_BF = jnp.bfloat16

_LOG2E = 1.4426950408889634
_LN2 = 0.6931471805599453


def _ssp_scaled(z):
    t = jnp.exp2(-jnp.abs(z))
    return (jnp.log2(1.0 + t) - 1.0 - jnp.minimum(z, 0.0)) * _LN2


def _y_kernel(x_ref, Wi_ref, y_ref):
    b = pl.program_id(0)

    @pl.when(b < _B)
    def _():
        yv = jnp.dot(x_ref[0].astype(_BF), Wi_ref[:],
                     preferred_element_type=jnp.float32)
        wl = lax.bitcast_convert_type(yv[:, :_NF // 2], jnp.uint32)
        wh = lax.bitcast_convert_type(yv[:, _NF // 2:], jnp.uint32)
        wl = (wl + jnp.uint32(0x8000)) >> jnp.uint32(16)
        wh = (wh + jnp.uint32(0x8000)) & jnp.uint32(0xFFFF0000)
        y_ref[0] = lax.bitcast_convert_type(wh | wl, jnp.int32)

    @pl.when(b == _B)
    def _():
        y_ref[0] = jnp.zeros((_N, _NF // 2), jnp.int32)


_SC_INFO = plsc.get_sparse_core_info()
_NW = _SC_INFO.num_cores * _SC_INFO.num_subcores
_PER_W = _NE // _NW
_CH = 64
_NCH = _PER_W // _CH


def _sc_gather(table_hbm, idx_hbm, out_hbm, idx_v, rows0, rows1, sem0, sem1):
    wid = lax.axis_index("s") * _SC_INFO.num_cores + lax.axis_index("c")
    base = wid * _PER_W
    pltpu.sync_copy(idx_hbm.at[pl.ds(base, _PER_W)], idx_v)

    bufs = (rows0, rows1)
    sems = (sem0, sem1)
    cp_prev = None
    for c in range(_NCH):
        cp = pltpu.async_copy(table_hbm.at[idx_v.at[pl.ds(c * _CH, _CH)]],
                              bufs[c % 2], sems[c % 2])
        if cp_prev is not None:
            cp_prev.wait()
            pltpu.sync_copy(bufs[(c - 1) % 2],
                            out_hbm.at[pl.ds(base + (c - 1) * _CH, _CH)])
        cp_prev = cp
    cp_prev.wait()
    pltpu.sync_copy(bufs[(_NCH - 1) % 2],
                    out_hbm.at[pl.ds(base + (_NCH - 1) * _CH, _CH)])


def _comb_kernel(ynbh_ref, f_ref, W1_ref, b1_ref, W2_ref, b2_ref,
                 Wf_ref, bf_ref, Wd_ref, bd_ref, out_ref):
    f = f_ref[0].reshape(_E, _SB)
    h = _ssp_scaled(jnp.dot(f.astype(_BF), W1_ref[:],
                            preferred_element_type=jnp.float32) + b1_ref[:])
    wf = jnp.dot(h.astype(_BF), W2_ref[:],
                 preferred_element_type=jnp.float32) + b2_ref[:]
    w = lax.bitcast_convert_type(ynbh_ref[0, 0], jnp.uint32)
    y_lo = lax.bitcast_convert_type(w << jnp.uint32(16), jnp.float32)
    y_hi = lax.bitcast_convert_type(w & jnp.uint32(0xFFFF0000), jnp.float32)
    agg_lo = (y_lo * wf[:, :_NF // 2]).reshape(_BN, _NBH, _NF // 2).sum(axis=1)
    agg_hi = (y_hi * wf[:, _NF // 2:]).reshape(_BN, _NBH, _NF // 2).sum(axis=1)
    agg = jnp.concatenate([agg_lo, agg_hi], axis=1)
    v = _ssp_scaled(jnp.dot(agg.astype(_BF), Wf_ref[:],
                            preferred_element_type=jnp.float32) + bf_ref[:])
    out_ref[0] = jnp.dot(v.astype(_BF), Wd_ref[:],
                         preferred_element_type=jnp.float32) + bd_ref[:]


def kernel(x, r_ij, neighbors, neighbor_mask, f_ij,
           W1, b1, W2, b2, Wi, Wf, bf, Wd, bd):
    del r_ij

    y = pl.pallas_call(
        _y_kernel,
        grid=(_B + 1,),
        in_specs=[
            pl.BlockSpec((1, _N, _AB), lambda b: (jnp.minimum(b, _B - 1), 0, 0)),
            pl.BlockSpec((_AB, _NF), lambda b: (0, 0)),
        ],
        out_specs=pl.BlockSpec((1, _N, _NF // 2), lambda b: (b, 0, 0)),
        out_shape=jax.ShapeDtypeStruct((_B + 1, _N, _NF // 2), jnp.int32),
    )(x, Wi.astype(_BF))
    table = y.reshape((_B + 1) * _N, _NF // 2)

    boff = (jnp.arange(_B, dtype=jnp.int32) * _N)[:, None, None]
    gidx = jnp.where(neighbor_mask > 0, neighbors + boff, _B * _N)
    gidx = gidx.reshape(_NE)

    sc_gather = functools.partial(
        pl.kernel,
        mesh=plsc.VectorSubcoreMesh(core_axis_name="c", subcore_axis_name="s"),
        out_type=jax.ShapeDtypeStruct((_NE, _NF // 2), jnp.int32),
        scratch_types=[
            pltpu.VMEM((_PER_W,), jnp.int32),
            pltpu.VMEM((_CH, _NF // 2), jnp.int32),
            pltpu.VMEM((_CH, _NF // 2), jnp.int32),
            pltpu.SemaphoreType.DMA,
            pltpu.SemaphoreType.DMA,
        ],
    )(_sc_gather)
    y_nbh = sc_gather(table, gidx)
    ynbh_r = y_nbh.reshape(_B, _NBLK, _E, _NF // 2)

    full = lambda shape: pl.BlockSpec(shape, lambda b, nb: (0,) * len(shape))

    out = pl.pallas_call(
        _comb_kernel,
        grid=(_B, _NBLK),
        in_specs=[
            pl.BlockSpec((1, 1, _E, _NF // 2), lambda b, nb: (b, nb, 0, 0)),
            pl.BlockSpec((1, _BN, _NBH, _SB), lambda b, nb: (b, nb, 0, 0)),
            full((_SB, _NF)),
            full((1, _NF)),
            full((_NF, _NF)),
            full((1, _NF)),
            full((_NF, _AB)),
            full((1, _AB)),
            full((_AB, _AB)),
            full((1, _AB)),
        ],
        out_specs=pl.BlockSpec((1, _BN, _AB), lambda b, nb: (b, nb, 0)),
        out_shape=jax.ShapeDtypeStruct((_B, _N, _AB), jnp.float32),
        compiler_params=pltpu.CompilerParams(
            dimension_semantics=("parallel", "parallel"),
        ),
    )(ynbh_r, f_ij,
      (W1 * -_LOG2E).astype(_BF), (b1 * -_LOG2E).reshape(1, _NF),
      W2.astype(_BF), b2.reshape(1, _NF),
      (Wf * -_LOG2E).astype(_BF), (bf * -_LOG2E).reshape(1, _AB),
      Wd.astype(_BF), bd.reshape(1, _AB))
    return out
